# baseline (speedup 1.0000x reference)
import jax
import jax.numpy as jnp
from jax.experimental import pallas as pl
from jax.experimental.pallas import tpu as pltpu

N_E = 1024
E_DIM = 64
BETA = 0.25
B = 16
P = 1024
N_TOK = B * P
TILE = 2048
GRID = N_TOK // TILE


def _vq_kernel(zf_ref, e_ref, e2_ref, esq_ref, zsq_ref,
               enc_ref, zq_ref, idx_ref, loss_ref, cnt_ref):
    zf = zf_ref[...]
    e = e_ref[...]
    e2 = e2_ref[...]
    e_sq = esq_ref[...]
    z_sq = zsq_ref[0]

    scores_n2 = jax.lax.dot_general(e2, zf, (((1,), (1,)), ((), ())))
    dist = (e_sq + z_sq) + scores_n2

    iota_sub = jax.lax.broadcasted_iota(jnp.int32, (N_E, TILE), 0).astype(
        jnp.float32)
    m = jnp.min(dist, axis=0, keepdims=True)
    idx_row = jnp.min(jnp.where(dist == m, iota_sub, float(N_E)), axis=0,
                      keepdims=True)

    idx_ref[0, 0] = idx_row[0].astype(jnp.int32)

    iota_lane = jax.lax.broadcasted_iota(jnp.int32, (TILE, N_E), 1).astype(
        jnp.float32)
    idx_col = idx_row.reshape(TILE, 1)
    enc = (iota_lane == idx_col).astype(jnp.float32)
    enc_ref[0] = enc

    zq = jax.lax.dot_general(enc, e, (((1,), (0,)), ((), ())))
    diff = zq - zf
    zq_ref[...] = zf + diff

    loss_ref[...] = jnp.sum(diff * diff).reshape(1, 1, 1)
    cnt_ref[0] = jnp.sum(enc, axis=0, keepdims=True)


def kernel(z, embedding_weight):
    z_flat = jnp.transpose(z.reshape(B, E_DIM, P), (0, 2, 1)).reshape(
        N_TOK, E_DIM)
    z_sq = jnp.sum(z_flat ** 2, axis=1).reshape(GRID, 1, TILE)
    e2 = -2.0 * embedding_weight
    e_sq = jnp.sum(embedding_weight ** 2, axis=1).reshape(N_E, 1)
    out_shapes = (
        jax.ShapeDtypeStruct((GRID, TILE, N_E), jnp.float32),
        jax.ShapeDtypeStruct((N_TOK, E_DIM), jnp.float32),
        jax.ShapeDtypeStruct((GRID, 1, TILE), jnp.int32),
        jax.ShapeDtypeStruct((GRID, 1, 1), jnp.float32),
        jax.ShapeDtypeStruct((GRID, 1, N_E), jnp.float32),
    )
    enc, zq, idx, loss_p, cnt_p = pl.pallas_call(
        _vq_kernel,
        grid=(GRID,),
        in_specs=[
            pl.BlockSpec((TILE, E_DIM), lambda b: (b, 0)),
            pl.BlockSpec((N_E, E_DIM), lambda b: (0, 0)),
            pl.BlockSpec((N_E, E_DIM), lambda b: (0, 0)),
            pl.BlockSpec((N_E, 1), lambda b: (0, 0)),
            pl.BlockSpec((1, 1, TILE), lambda b: (b, 0, 0)),
        ],
        out_specs=(
            pl.BlockSpec((1, TILE, N_E), lambda b: (b, 0, 0)),
            pl.BlockSpec((TILE, E_DIM), lambda b: (b, 0)),
            pl.BlockSpec((1, 1, TILE), lambda b: (b, 0, 0)),
            pl.BlockSpec((1, 1, 1), lambda b: (b, 0, 0)),
            pl.BlockSpec((1, 1, N_E), lambda b: (b, 0, 0)),
        ),
        out_shape=out_shapes,
        compiler_params=pltpu.CompilerParams(
            dimension_semantics=("arbitrary",),
        ),
    )(z_flat, embedding_weight, e2, e_sq, z_sq)
    loss = jnp.sum(loss_p) / (N_TOK * E_DIM) * BETA
    avg_probs = jnp.sum(cnt_p[:, 0, :], axis=0) / N_TOK
    perplexity = jnp.exp(-jnp.sum(avg_probs * jnp.log(avg_probs + 1e-10)))
    min_encodings = enc.reshape(N_TOK, N_E)
    min_encoding_indices = idx.reshape(N_TOK)
    z_q_st = jnp.transpose(zq.reshape(B, P, E_DIM), (0, 2, 1)).reshape(z.shape)
    return (loss, z_q_st, perplexity, min_encodings, min_encoding_indices)


# speedup vs baseline: 1.0904x; 1.0904x over previous
import jax
import jax.numpy as jnp
from jax.experimental import pallas as pl
from jax.experimental.pallas import tpu as pltpu

N_E = 1024
E_DIM = 64
BETA = 0.25
B = 16
P = 1024
N_TOK = B * P
TILE = 2048
GRID = N_TOK // TILE


def _vq_kernel(zf_ref, e_ref, e2_ref, esq_ref,
               enc_ref, zq_ref, idx_ref, loss_ref, cnt_ref):
    zf = zf_ref[...]
    e = e_ref[...]
    e2 = e2_ref[...]
    e_sq = esq_ref[...]
    z_sq = jnp.sum(zf * zf, axis=1)[None, :]

    scores_n2 = jax.lax.dot_general(e2, zf, (((1,), (1,)), ((), ())))
    dist = (e_sq + z_sq) + scores_n2

    iota_sub = jax.lax.broadcasted_iota(jnp.int32, (N_E, TILE), 0).astype(
        jnp.float32)
    m = jnp.min(dist, axis=0, keepdims=True)
    idx_row = jnp.min(jnp.where(dist == m, iota_sub, float(N_E)), axis=0,
                      keepdims=True)

    idx_ref[0, 0] = idx_row[0].astype(jnp.int32)

    iota_lane = jax.lax.broadcasted_iota(jnp.int32, (TILE, N_E), 1).astype(
        jnp.float32)
    idx_col = idx_row.reshape(TILE, 1)
    enc = (iota_lane == idx_col).astype(jnp.float32)
    enc_ref[0] = enc

    zq = jax.lax.dot_general(enc, e, (((1,), (0,)), ((), ())))
    diff = zq - zf
    zq_ref[...] = zf + diff

    loss_ref[...] = jnp.sum(diff * diff).reshape(1, 1, 1)
    cnt_ref[0] = jnp.sum(enc, axis=0, keepdims=True)


def kernel(z, embedding_weight):
    z_flat = jnp.transpose(z.reshape(B, E_DIM, P), (0, 2, 1)).reshape(
        N_TOK, E_DIM)
    e2 = -2.0 * embedding_weight
    e_sq = jnp.sum(embedding_weight ** 2, axis=1).reshape(N_E, 1)
    out_shapes = (
        jax.ShapeDtypeStruct((GRID, TILE, N_E), jnp.float32),
        jax.ShapeDtypeStruct((N_TOK, E_DIM), jnp.float32),
        jax.ShapeDtypeStruct((GRID, 1, TILE), jnp.int32),
        jax.ShapeDtypeStruct((GRID, 1, 1), jnp.float32),
        jax.ShapeDtypeStruct((GRID, 1, N_E), jnp.float32),
    )
    enc, zq, idx, loss_p, cnt_p = pl.pallas_call(
        _vq_kernel,
        grid=(GRID,),
        in_specs=[
            pl.BlockSpec((TILE, E_DIM), lambda b: (b, 0)),
            pl.BlockSpec((N_E, E_DIM), lambda b: (0, 0)),
            pl.BlockSpec((N_E, E_DIM), lambda b: (0, 0)),
            pl.BlockSpec((N_E, 1), lambda b: (0, 0)),
        ],
        out_specs=(
            pl.BlockSpec((1, TILE, N_E), lambda b: (b, 0, 0)),
            pl.BlockSpec((TILE, E_DIM), lambda b: (b, 0)),
            pl.BlockSpec((1, 1, TILE), lambda b: (b, 0, 0)),
            pl.BlockSpec((1, 1, 1), lambda b: (b, 0, 0)),
            pl.BlockSpec((1, 1, N_E), lambda b: (b, 0, 0)),
        ),
        out_shape=out_shapes,
        compiler_params=pltpu.CompilerParams(
            dimension_semantics=("arbitrary",),
        ),
    )(z_flat, embedding_weight, e2, e_sq)
    loss = jnp.sum(loss_p) / (N_TOK * E_DIM) * BETA
    avg_probs = jnp.sum(cnt_p[:, 0, :], axis=0) / N_TOK
    perplexity = jnp.exp(-jnp.sum(avg_probs * jnp.log(avg_probs + 1e-10)))
    min_encodings = enc.reshape(N_TOK, N_E)
    min_encoding_indices = idx.reshape(N_TOK)
    z_q_st = jnp.transpose(zq.reshape(B, P, E_DIM), (0, 2, 1)).reshape(z.shape)
    return (loss, z_q_st, perplexity, min_encodings, min_encoding_indices)


# speedup vs baseline: 1.1332x; 1.0392x over previous
import jax
import jax.numpy as jnp
from jax.experimental import pallas as pl
from jax.experimental.pallas import tpu as pltpu

N_E = 1024
E_DIM = 64
BETA = 0.25
B = 16
P = 1024
N_TOK = B * P
TILE = 2048
GRID = N_TOK // TILE


def _vq_kernel(zf_ref, e_ref, enc_ref, zq_ref, idx_ref, loss_ref, cnt_ref,
               e2_scr, esq_scr):
    step = pl.program_id(0)
    zf = zf_ref[...]
    e = e_ref[...]

    @pl.when(step == 0)
    def _init():
        e2_scr[...] = -2.0 * e
        esq_scr[...] = jnp.sum(e * e, axis=1, keepdims=True)

    scores_n2 = jax.lax.dot_general(e2_scr[...], zf,
                                    (((1,), (1,)), ((), ())))
    z_sq = jnp.sum(zf * zf, axis=1)
    dist = (esq_scr[...] + z_sq[None, :]) + scores_n2

    iota_sub = jax.lax.broadcasted_iota(jnp.int32, (N_E, TILE), 0).astype(
        jnp.float32)
    m = jnp.min(dist, axis=0, keepdims=True)
    idx_row = jnp.min(jnp.where(dist == m, iota_sub, float(N_E)), axis=0,
                      keepdims=True)

    idx_ref[0, 0] = idx_row[0].astype(jnp.int32)

    iota_lane = jax.lax.broadcasted_iota(jnp.int32, (TILE, N_E), 1).astype(
        jnp.float32)
    idx_col = idx_row.reshape(TILE, 1)
    enc = (iota_lane == idx_col).astype(jnp.float32)
    enc_ref[0] = enc

    zq = jax.lax.dot_general(enc, e, (((1,), (0,)), ((), ())))
    diff = zq - zf
    zq_ref[...] = zf + diff

    loss_ref[...] = jnp.sum(diff * diff).reshape(1, 1, 1)
    cnt_ref[0] = jnp.sum(enc, axis=0, keepdims=True)


def kernel(z, embedding_weight):
    z_flat = jnp.transpose(z.reshape(B, E_DIM, P), (0, 2, 1)).reshape(
        N_TOK, E_DIM)
    out_shapes = (
        jax.ShapeDtypeStruct((GRID, TILE, N_E), jnp.float32),
        jax.ShapeDtypeStruct((N_TOK, E_DIM), jnp.float32),
        jax.ShapeDtypeStruct((GRID, 1, TILE), jnp.int32),
        jax.ShapeDtypeStruct((GRID, 1, 1), jnp.float32),
        jax.ShapeDtypeStruct((GRID, 1, N_E), jnp.float32),
    )
    enc, zq, idx, loss_p, cnt_p = pl.pallas_call(
        _vq_kernel,
        grid=(GRID,),
        in_specs=[
            pl.BlockSpec((TILE, E_DIM), lambda b: (b, 0)),
            pl.BlockSpec((N_E, E_DIM), lambda b: (0, 0)),
        ],
        out_specs=(
            pl.BlockSpec((1, TILE, N_E), lambda b: (b, 0, 0)),
            pl.BlockSpec((TILE, E_DIM), lambda b: (b, 0)),
            pl.BlockSpec((1, 1, TILE), lambda b: (b, 0, 0)),
            pl.BlockSpec((1, 1, 1), lambda b: (b, 0, 0)),
            pl.BlockSpec((1, 1, N_E), lambda b: (b, 0, 0)),
        ),
        out_shape=out_shapes,
        scratch_shapes=[
            pltpu.VMEM((N_E, E_DIM), jnp.float32),
            pltpu.VMEM((N_E, 1), jnp.float32),
        ],
        compiler_params=pltpu.CompilerParams(
            dimension_semantics=("arbitrary",),
        ),
    )(z_flat, embedding_weight)
    loss = jnp.sum(loss_p) / (N_TOK * E_DIM) * BETA
    avg_probs = jnp.sum(cnt_p[:, 0, :], axis=0) / N_TOK
    perplexity = jnp.exp(-jnp.sum(avg_probs * jnp.log(avg_probs + 1e-10)))
    min_encodings = enc.reshape(N_TOK, N_E)
    min_encoding_indices = idx.reshape(N_TOK)
    z_q_st = jnp.transpose(zq.reshape(B, P, E_DIM), (0, 2, 1)).reshape(z.shape)
    return (loss, z_q_st, perplexity, min_encodings, min_encoding_indices)


# speedup vs baseline: 1.1518x; 1.0165x over previous
import jax
import jax.numpy as jnp
from jax.experimental import pallas as pl
from jax.experimental.pallas import tpu as pltpu

N_E = 1024
E_DIM = 64
BETA = 0.25
B = 16
P = 1024
N_TOK = B * P
TILE = 2048
GRID = N_TOK // TILE


def _vq_kernel(zf_ref, e_ref, enc_ref, zq_ref, idx_ref, loss_ref, cnt_ref):
    zf = zf_ref[...]
    e = e_ref[...]

    e2 = -2.0 * e
    scores_n2 = jax.lax.dot_general(e2, zf, (((1,), (1,)), ((), ())))
    z_sq = jnp.sum(zf * zf, axis=1)
    e_sq = jnp.sum(e * e, axis=1)
    dist = (e_sq[:, None] + z_sq[None, :]) + scores_n2

    iota_sub = jax.lax.broadcasted_iota(jnp.int32, (N_E, TILE), 0).astype(
        jnp.float32)
    m = jnp.min(dist, axis=0, keepdims=True)
    idx_row = jnp.min(jnp.where(dist == m, iota_sub, float(N_E)), axis=0,
                      keepdims=True)

    idx_ref[0, 0] = idx_row[0].astype(jnp.int32)

    iota_lane = jax.lax.broadcasted_iota(jnp.int32, (TILE, N_E), 1).astype(
        jnp.float32)
    idx_col = idx_row.reshape(TILE, 1)
    enc = (iota_lane == idx_col).astype(jnp.float32)
    enc_ref[0] = enc

    zq = jax.lax.dot_general(enc, e, (((1,), (0,)), ((), ())))
    diff = zq - zf
    zq_ref[...] = zf + diff

    loss_ref[...] = jnp.sum(diff * diff).reshape(1, 1, 1)
    cnt_ref[0] = jnp.sum(enc, axis=0, keepdims=True)


def kernel(z, embedding_weight):
    z_flat = jnp.transpose(z.reshape(B, E_DIM, P), (0, 2, 1)).reshape(
        N_TOK, E_DIM)
    out_shapes = (
        jax.ShapeDtypeStruct((GRID, TILE, N_E), jnp.float32),
        jax.ShapeDtypeStruct((N_TOK, E_DIM), jnp.float32),
        jax.ShapeDtypeStruct((GRID, 1, TILE), jnp.int32),
        jax.ShapeDtypeStruct((GRID, 1, 1), jnp.float32),
        jax.ShapeDtypeStruct((GRID, 1, N_E), jnp.float32),
    )
    enc, zq, idx, loss_p, cnt_p = pl.pallas_call(
        _vq_kernel,
        grid=(GRID,),
        in_specs=[
            pl.BlockSpec((TILE, E_DIM), lambda b: (b, 0)),
            pl.BlockSpec((N_E, E_DIM), lambda b: (0, 0)),
        ],
        out_specs=(
            pl.BlockSpec((1, TILE, N_E), lambda b: (b, 0, 0)),
            pl.BlockSpec((TILE, E_DIM), lambda b: (b, 0)),
            pl.BlockSpec((1, 1, TILE), lambda b: (b, 0, 0)),
            pl.BlockSpec((1, 1, 1), lambda b: (b, 0, 0)),
            pl.BlockSpec((1, 1, N_E), lambda b: (b, 0, 0)),
        ),
        out_shape=out_shapes,
        compiler_params=pltpu.CompilerParams(
            dimension_semantics=("arbitrary",),
        ),
    )(z_flat, embedding_weight)
    loss = jnp.sum(loss_p) / (N_TOK * E_DIM) * BETA
    avg_probs = jnp.sum(cnt_p[:, 0, :], axis=0) / N_TOK
    perplexity = jnp.exp(-jnp.sum(avg_probs * jnp.log(avg_probs + 1e-10)))
    min_encodings = enc.reshape(N_TOK, N_E)
    min_encoding_indices = idx.reshape(N_TOK)
    z_q_st = jnp.transpose(zq.reshape(B, P, E_DIM), (0, 2, 1)).reshape(z.shape)
    return (loss, z_q_st, perplexity, min_encodings, min_encoding_indices)
